# baseline (speedup 1.0000x reference)
import functools

import jax
import jax.numpy as jnp
from jax import lax
from jax.experimental import pallas as pl
from jax.experimental.pallas import tpu as pltpu
from jax.experimental.pallas import tpu_sc as plsc

OBS = 64
SEQ_LEN = 200
SKILL_ROWS = 2
OUT_ROWS_PER_B = SEQ_LEN + SKILL_ROWS
BATCH = 4096
NUM_WORKERS = 32
B_PER_W = BATCH // NUM_WORKERS
LANES = 16


def _sc_kernel(seq_rows, skill_rows, obj_idx, obj_embed, out_rows,
               idx_v, gidx_v, sidx_v, skidx_v, rows_v, skill_v, emb_v, sem):
    wid = lax.axis_index("s") * 2 + lax.axis_index("c")
    base = wid * B_PER_W

    pltpu.sync_copy(obj_idx.at[pl.ds(base, B_PER_W)], idx_v)
    pltpu.sync_copy(obj_embed, emb_v)
    pltpu.sync_copy(skill_rows.at[pl.ds(base * SKILL_ROWS,
                                        B_PER_W * SKILL_ROWS)], skill_v)

    iota = lax.iota(jnp.int32, LANES)
    for i in range(B_PER_W // LANES):
        bvec = base + i * LANES + iota
        ivec = idx_v[pl.ds(i * LANES, LANES)]
        gidx_v[pl.ds(i * LANES, LANES)] = bvec * SEQ_LEN + ivec
        sidx_v[pl.ds(i * LANES, LANES)] = bvec * OUT_ROWS_PER_B + ivec
    for i in range(B_PER_W * SKILL_ROWS // LANES):
        kvec = i * LANES + iota
        bvec = base + lax.shift_right_logical(kvec, 1)
        tvec = lax.bitwise_and(kvec, 1)
        skidx_v[pl.ds(i * LANES, LANES)] = (
            bvec * OUT_ROWS_PER_B + SEQ_LEN + tvec)

    pltpu.async_copy(seq_rows.at[gidx_v], rows_v, sem).wait()

    evecs = [emb_v[pl.ds(j * LANES, LANES)] for j in range(OBS // LANES)]

    def add_body(r, carry):
        for j in range(OBS // LANES):
            rows_v[r, pl.ds(j * LANES, LANES)] = (
                rows_v[r, pl.ds(j * LANES, LANES)] + evecs[j])
        return carry

    lax.fori_loop(0, B_PER_W, add_body, 0)

    def copy_body(i, carry):
        b = base + i
        pltpu.sync_copy(
            seq_rows.at[pl.ds(b * SEQ_LEN, SEQ_LEN)],
            out_rows.at[pl.ds(b * OUT_ROWS_PER_B, SEQ_LEN)])
        return carry

    lax.fori_loop(0, B_PER_W, copy_body, 0)

    pltpu.async_copy(skill_v, out_rows.at[skidx_v], sem).wait()
    pltpu.async_copy(rows_v, out_rows.at[sidx_v], sem).wait()


@jax.jit
def kernel(seq, skill, obj_idx, obj_embed):
    batch, seq_len, obs = seq.shape
    seq_rows = seq.reshape(batch * seq_len, obs)
    skill_rows = skill.reshape(batch * SKILL_ROWS, obs)
    obj_idx = obj_idx.astype(jnp.int32)

    mesh = plsc.VectorSubcoreMesh(core_axis_name="c", subcore_axis_name="s")
    out_rows = pl.kernel(
        _sc_kernel,
        out_type=jax.ShapeDtypeStruct((batch * OUT_ROWS_PER_B, obs),
                                      jnp.float32),
        mesh=mesh,
        compiler_params=pltpu.CompilerParams(use_tc_tiling_on_sc=False),
        scratch_types=[
            pltpu.VMEM((B_PER_W,), jnp.int32),
            pltpu.VMEM((B_PER_W,), jnp.int32),
            pltpu.VMEM((B_PER_W,), jnp.int32),
            pltpu.VMEM((B_PER_W * SKILL_ROWS,), jnp.int32),
            pltpu.VMEM((B_PER_W, OBS), jnp.float32),
            pltpu.VMEM((B_PER_W * SKILL_ROWS, OBS), jnp.float32),
            pltpu.VMEM((OBS,), jnp.float32),
            pltpu.SemaphoreType.DMA,
        ],
    )(seq_rows, skill_rows, obj_idx, obj_embed)
    return out_rows.reshape(batch, seq_len * obs + SKILL_ROWS * obs)


# speedup vs baseline: 8.3647x; 8.3647x over previous
import jax
import jax.numpy as jnp
from jax import lax
from jax.experimental import pallas as pl
from jax.experimental.pallas import tpu as pltpu
from jax.experimental.pallas import tpu_sc as plsc

OBS = 64
SEQ_LEN = 200
SKILL_ROWS = 2
OUT_ROWS_PER_B = SEQ_LEN + SKILL_ROWS
BATCH = 4096
NUM_WORKERS = 32
B_PER_W = BATCH // NUM_WORKERS
LANES = 16
CHUNK_B = 4
NCHUNK = B_PER_W // CHUNK_B
BUF_ROWS = CHUNK_B * OUT_ROWS_PER_B


def _sc_kernel(seq_rows, skill_rows, obj_idx, obj_embed, out_rows,
               idx_v, gidx_v, sidx_v, skidx_v, rows_v, skill_v, emb_v, buf,
               semi0, semi1, semo0, semo1, semg):
    wid = lax.axis_index("s") * 2 + lax.axis_index("c")
    base = wid * B_PER_W
    sem_in = [semi0, semi1]
    sem_out = [semo0, semo1]

    pltpu.sync_copy(obj_idx.at[pl.ds(base, B_PER_W)], idx_v)
    pltpu.sync_copy(obj_embed, emb_v)
    pltpu.sync_copy(skill_rows.at[pl.ds(base * SKILL_ROWS,
                                        B_PER_W * SKILL_ROWS)], skill_v)

    iota = lax.iota(jnp.int32, LANES)
    for i in range(B_PER_W // LANES):
        bvec = base + i * LANES + iota
        ivec = idx_v[pl.ds(i * LANES, LANES)]
        gidx_v[pl.ds(i * LANES, LANES)] = bvec * SEQ_LEN + ivec
        sidx_v[pl.ds(i * LANES, LANES)] = bvec * OUT_ROWS_PER_B + ivec
    for i in range(B_PER_W * SKILL_ROWS // LANES):
        kvec = i * LANES + iota
        bvec = base + lax.shift_right_logical(kvec, 1)
        tvec = lax.bitwise_and(kvec, 1)
        skidx_v[pl.ds(i * LANES, LANES)] = (
            bvec * OUT_ROWS_PER_B + SEQ_LEN + tvec)

    gather = pltpu.async_copy(seq_rows.at[gidx_v], rows_v, semg)

    def start_in(k, c):
        b0 = base + c * CHUNK_B
        for i in range(CHUNK_B):
            pltpu.make_async_copy(
                seq_rows.at[pl.ds((b0 + i) * SEQ_LEN, SEQ_LEN)],
                buf.at[k, pl.ds(i * OUT_ROWS_PER_B, SEQ_LEN)],
                sem_in[k]).start()

    def wait_in(k):
        pltpu.make_async_copy(
            seq_rows.at[pl.ds(0, CHUNK_B * SEQ_LEN)],
            buf.at[k, pl.ds(0, CHUNK_B * SEQ_LEN)],
            sem_in[k]).wait()

    def start_out(k, c):
        pltpu.make_async_copy(
            buf.at[k],
            out_rows.at[pl.ds((base + c * CHUNK_B) * OUT_ROWS_PER_B,
                              BUF_ROWS)],
            sem_out[k]).start()

    def wait_out(k):
        pltpu.make_async_copy(
            buf.at[k],
            out_rows.at[pl.ds(base * OUT_ROWS_PER_B, BUF_ROWS)],
            sem_out[k]).wait()

    start_in(0, 0)
    start_in(1, 1)

    def body(it, carry):
        for k in (0, 1):
            c = it * 2 + k
            wait_in(k)
            start_out(k, c)
        for k in (0, 1):
            c = it * 2 + k
            wait_out(k)

            @pl.when(c + 2 < NCHUNK)
            def _():
                start_in(k, c + 2)
        return carry

    lax.fori_loop(0, NCHUNK // 2, body, 0)

    gather.wait()
    evecs = [emb_v[pl.ds(j * LANES, LANES)] for j in range(OBS // LANES)]

    def add_body(r, carry):
        for j in range(OBS // LANES):
            rows_v[r, pl.ds(j * LANES, LANES)] = (
                rows_v[r, pl.ds(j * LANES, LANES)] + evecs[j])
        return carry

    lax.fori_loop(0, B_PER_W, add_body, 0)

    pltpu.async_copy(skill_v, out_rows.at[skidx_v], semg).wait()
    pltpu.async_copy(rows_v, out_rows.at[sidx_v], semg).wait()


@jax.jit
def kernel(seq, skill, obj_idx, obj_embed):
    batch, seq_len, obs = seq.shape
    seq_rows = seq.reshape(batch * seq_len, obs)
    skill_rows = skill.reshape(batch * SKILL_ROWS, obs)
    obj_idx = obj_idx.astype(jnp.int32)

    mesh = plsc.VectorSubcoreMesh(core_axis_name="c", subcore_axis_name="s")
    out_rows = pl.kernel(
        _sc_kernel,
        out_type=jax.ShapeDtypeStruct((batch * OUT_ROWS_PER_B, obs),
                                      jnp.float32),
        mesh=mesh,
        compiler_params=pltpu.CompilerParams(use_tc_tiling_on_sc=False),
        scratch_types=[
            pltpu.VMEM((B_PER_W,), jnp.int32),
            pltpu.VMEM((B_PER_W,), jnp.int32),
            pltpu.VMEM((B_PER_W,), jnp.int32),
            pltpu.VMEM((B_PER_W * SKILL_ROWS,), jnp.int32),
            pltpu.VMEM((B_PER_W, OBS), jnp.float32),
            pltpu.VMEM((B_PER_W * SKILL_ROWS, OBS), jnp.float32),
            pltpu.VMEM((OBS,), jnp.float32),
            pltpu.VMEM((2, BUF_ROWS, OBS), jnp.float32),
            pltpu.SemaphoreType.DMA,
            pltpu.SemaphoreType.DMA,
            pltpu.SemaphoreType.DMA,
            pltpu.SemaphoreType.DMA,
            pltpu.SemaphoreType.DMA,
        ],
    )(seq_rows, skill_rows, obj_idx, obj_embed)
    return out_rows.reshape(batch, seq_len * obs + SKILL_ROWS * obs)
